# baseline (speedup 1.0000x reference)
import functools

import jax
import jax.numpy as jnp
from jax import lax
from jax.experimental import pallas as pl
from jax.experimental.pallas import tpu as pltpu
from jax.experimental.pallas import tpu_sc as plsc

DN = 128
NLAYERS = 4

BE = 2560

NC = 2
NS = 16
NW = NC * NS
KCH = 128

GCH = 640
GSUB = GCH // KCH


def _gelu(v):
    return jax.nn.gelu(v, approximate=True)


def _sc_gather(xt, posp, src2, dst2):
    n = xt.shape[0]
    e = src2.shape[0] * src2.shape[1] * src2.shape[2]
    nch = e // GCH
    iters = (nch + NW - 1) // NW
    mesh = plsc.VectorSubcoreMesh(core_axis_name="c", subcore_axis_name="s")

    def body(xt_hbm, posp_hbm, src_hbm, dst_hbm,
             gxs_hbm, gxd_hbm, r_hbm,
             idx_s, idx_d, rows_s0, rows_s1, rows_d0, rows_d1,
             rbuf, px_v, py_v, pz_v,
             sem_s0, sem_s1, sem_d0, sem_d1):
        wid = lax.axis_index("s") * NC + lax.axis_index("c")
        rows_s = (rows_s0, rows_s1)
        rows_d = (rows_d0, rows_d1)
        sem_s = (sem_s0, sem_s1)
        sem_d = (sem_d0, sem_d1)
        pltpu.sync_copy(posp_hbm.at[0], px_v)
        pltpu.sync_copy(posp_hbm.at[1], py_v)
        pltpu.sync_copy(posp_hbm.at[2], pz_v)
        for r in range(4, 8):
            for j in range(GCH // 16):
                rbuf[r, pl.ds(j * 16, 16)] = jnp.zeros((16,), jnp.float32)

        def step(i, carry):
            c = wid + i * NW

            @pl.when(c < nch)
            def _():
                base = c * GCH
                pltpu.sync_copy(src_hbm.at[c], idx_s)
                pltpu.sync_copy(dst_hbm.at[c], idx_d)
                descs = [None, None]

                def _drain(sub):
                    b = sub % 2
                    descs[b][0].wait()
                    descs[b][1].wait()
                    pltpu.sync_copy(
                        rows_s[b], gxs_hbm.at[pl.ds(base + sub * KCH, KCH)])
                    pltpu.sync_copy(
                        rows_d[b], gxd_hbm.at[pl.ds(base + sub * KCH, KCH)])

                for sub in range(GSUB):
                    b = sub % 2
                    cs = pltpu.async_copy(xt_hbm.at[idx_s.at[sub]],
                                          rows_s[b], sem_s[b])
                    cd = pltpu.async_copy(xt_hbm.at[idx_d.at[sub]],
                                          rows_d[b], sem_d[b])
                    descs[b] = (cs, cd)
                    for j in range(KCH // 16):
                        col = sub * KCH + j * 16
                        iv_s = idx_s[sub, pl.ds(j * 16, 16)]
                        iv_d = idx_d[sub, pl.ds(j * 16, 16)]
                        relx = (plsc.load_gather(px_v, [iv_s])
                                - plsc.load_gather(px_v, [iv_d]))
                        rely = (plsc.load_gather(py_v, [iv_s])
                                - plsc.load_gather(py_v, [iv_d]))
                        relz = (plsc.load_gather(pz_v, [iv_s])
                                - plsc.load_gather(pz_v, [iv_d]))
                        d2 = relx * relx + rely * rely + relz * relz
                        rbuf[0, pl.ds(col, 16)] = relx
                        rbuf[1, pl.ds(col, 16)] = rely
                        rbuf[2, pl.ds(col, 16)] = relz
                        rbuf[3, pl.ds(col, 16)] = d2
                    if sub > 0:
                        _drain(sub - 1)
                _drain(GSUB - 1)
                pltpu.sync_copy(rbuf, r_hbm.at[:, pl.ds(base, GCH)])
            return carry

        lax.fori_loop(0, iters, step, 0)

    f = pl.kernel(
        body,
        out_type=[jax.ShapeDtypeStruct((e, DN), jnp.float32),
                  jax.ShapeDtypeStruct((e, DN), jnp.float32),
                  jax.ShapeDtypeStruct((8, e), jnp.float32)],
        mesh=mesh,
        compiler_params=pltpu.CompilerParams(needs_layout_passes=False),
        scratch_types=[pltpu.VMEM((GSUB, KCH), jnp.int32),
                       pltpu.VMEM((GSUB, KCH), jnp.int32),
                       pltpu.VMEM((KCH, DN), jnp.float32),
                       pltpu.VMEM((KCH, DN), jnp.float32),
                       pltpu.VMEM((KCH, DN), jnp.float32),
                       pltpu.VMEM((KCH, DN), jnp.float32),
                       pltpu.VMEM((8, GCH), jnp.float32),
                       pltpu.VMEM((n,), jnp.float32),
                       pltpu.VMEM((n,), jnp.float32),
                       pltpu.VMEM((n,), jnp.float32),
                       pltpu.SemaphoreType.DMA,
                       pltpu.SemaphoreType.DMA,
                       pltpu.SemaphoreType.DMA,
                       pltpu.SemaphoreType.DMA])
    return f(xt, posp, src2, dst2)


def _sc_scatter_m(m, dsti, zrow, n):
    e = m.shape[0]
    nch = e // KCH
    iters = (nch + NW - 1) // NW
    nun = n // 16
    zit = (nun + NS - 1) // NS
    mesh = plsc.VectorSubcoreMesh(core_axis_name="c", subcore_axis_name="s")

    def body(m_hbm, dst_hbm, zr_hbm, aggm_hbm,
             idx_v, rows_v, zb, accm, sem):
        cid = lax.axis_index("c")
        sid = lax.axis_index("s")
        wid = sid * NC + cid

        pltpu.sync_copy(zr_hbm, zb)

        def zstep(k, carry):
            u = sid + k * NS

            @pl.when(u < nun)
            def _():
                pltpu.sync_copy(zb, accm.at[pl.ds(u * 16, 16)])
            return carry

        lax.fori_loop(0, zit, zstep, 0)
        plsc.subcore_barrier()

        def step(i, carry):
            c = wid + i * NW

            @pl.when(c < nch)
            def _():
                base = c * KCH
                pltpu.sync_copy(dst_hbm.at[pl.ds(base, KCH)], idx_v)
                pltpu.sync_copy(m_hbm.at[pl.ds(base, KCH)], rows_v)
                pltpu.sync_copy(rows_v, accm.at[idx_v], add=True)
            return carry

        lax.fori_loop(0, iters, step, 0)
        plsc.subcore_barrier()

        def ostep(k, carry):
            u = sid + k * NS

            @pl.when(u < nun)
            def _():
                pltpu.sync_copy(accm.at[pl.ds(u * 16, 16)],
                                aggm_hbm.at[cid, pl.ds(u * 16, 16)])
            return carry

        lax.fori_loop(0, zit, ostep, 0)

    f = pl.kernel(
        body,
        out_type=jax.ShapeDtypeStruct((NC, n, DN), jnp.float32),
        mesh=mesh,
        compiler_params=pltpu.CompilerParams(needs_layout_passes=False),
        scratch_types=[pltpu.VMEM((KCH,), jnp.int32),
                       pltpu.VMEM((KCH, DN), jnp.float32),
                       pltpu.VMEM((16, DN), jnp.float32),
                       pltpu.VMEM_SHARED((n, DN), jnp.float32),
                       pltpu.SemaphoreType.DMA])
    return f(m, dsti, zrow)


def _sc_scatter_p(p, dsti, zplane, n):
    e = p.shape[1]
    nch = e // GCH
    iters = (nch + NW - 1) // NW
    mesh = plsc.VectorSubcoreMesh(core_axis_name="c", subcore_axis_name="s")

    def body(p_hbm, dst_hbm, zp_hbm, aggp_hbm, idx_v, pbuf, accp):
        cid = lax.axis_index("c")
        sid = lax.axis_index("s")
        wid = sid * NC + cid

        pltpu.sync_copy(zp_hbm, accp)

        def step(i, carry):
            c = wid + i * NW

            @pl.when(c < nch)
            def _():
                base = c * GCH
                pltpu.sync_copy(dst_hbm.at[pl.ds(base, GCH)], idx_v)
                pltpu.sync_copy(p_hbm.at[:, pl.ds(base, GCH)], pbuf)
                for j in range(GCH // 16):
                    iv = idx_v[pl.ds(j * 16, 16)]
                    for k in range(4):
                        v = pbuf[k, pl.ds(j * 16, 16)]
                        plsc.addupdate_scatter(accp, [iv + k * n], v)
            return carry

        lax.fori_loop(0, iters, step, 0)
        pltpu.sync_copy(accp, aggp_hbm.at[wid])

    f = pl.kernel(
        body,
        out_type=jax.ShapeDtypeStruct((NW, 4 * n), jnp.float32),
        mesh=mesh,
        compiler_params=pltpu.CompilerParams(needs_layout_passes=False),
        scratch_types=[pltpu.VMEM((GCH,), jnp.int32),
                       pltpu.VMEM((8, GCH), jnp.float32),
                       pltpu.VMEM((4 * n,), jnp.float32)])
    return f(p, dsti, zplane)


def _edge_body(gxs_ref, gxd_ref, r_ref, ea_ref,
               we1_ref, be1_ref, we2_ref, be2_ref,
               wp1_ref, bp1_ref, wp2_ref, bp2_ref,
               m_out_ref, p_out_ref):
    xs = gxs_ref[...]
    xd = gxd_ref[...]
    relt = jnp.transpose(r_ref[...])
    d2 = relt[:, 3:4]
    s = jnp.concatenate([d2, d2 * 0.5, d2 * 0.25, d2 * 0.125], axis=1)
    fe = jnp.concatenate([jnp.sin(s), jnp.cos(s)], axis=1)
    w = we1_ref[...]
    bf = jnp.bfloat16
    f32 = jnp.float32
    dot = functools.partial(jnp.dot, preferred_element_type=f32)
    pre = (dot(xd.astype(bf), w[0:DN])
           + dot(xs.astype(bf), w[DN:2 * DN])
           + dot(fe.astype(bf), w[2 * DN:2 * DN + 8])
           + dot(ea_ref[...].astype(bf), w[2 * DN + 8:])
           + be1_ref[...])
    m1 = _gelu(pre)
    m = _gelu(dot(m1.astype(bf), we2_ref[...]) + be2_ref[...])
    p1 = _gelu(dot(m.astype(bf), wp1_ref[...]) + bp1_ref[...])
    p = jnp.sum(p1 * wp2_ref[...], axis=1, keepdims=True) + bp2_ref[...]
    nrows = xs.shape[0]
    pm = jnp.concatenate([relt[:, 0:3] * p,
                          jnp.ones((nrows, 1), jnp.float32),
                          jnp.zeros((nrows, 4), jnp.float32)], axis=1)
    m_out_ref[...] = m
    p_out_ref[...] = jnp.transpose(pm)


def _edge_mlp(gxs, gxd, r, ea, we1, be1, we2, be2, wp1, bp1, wp2r, bp2r):
    e = gxs.shape[0]
    full = lambda a: pl.BlockSpec(a.shape, lambda i: (0,) * a.ndim)
    blk = lambda w: pl.BlockSpec((BE, w), lambda i: (i, 0))
    pblk = pl.BlockSpec((8, BE), lambda i: (0, i))
    return pl.pallas_call(
        _edge_body,
        grid=(e // BE,),
        in_specs=[blk(DN), blk(DN), pblk, blk(16),
                  full(we1), full(be1), full(we2), full(be2),
                  full(wp1), full(bp1), full(wp2r), full(bp2r)],
        out_specs=[blk(DN), pblk],
        out_shape=[jax.ShapeDtypeStruct((e, DN), jnp.float32),
                   jax.ShapeDtypeStruct((8, e), jnp.float32)],
    )(gxs, gxd, r, ea, we1, be1, we2, be2, wp1, bp1, wp2r, bp2r)


def _node_body(x_ref, posp_ref, aggm_ref, aggp_ref,
               wn1_ref, bn1_ref, wn2_ref, bn2_ref,
               x_out_ref, posp_out_ref):
    x = x_ref[...]
    posp = posp_ref[...]
    aggm = jnp.sum(aggm_ref[...], axis=0)
    psum = jnp.sum(aggp_ref[...], axis=0)
    cnt = psum[3:4, :]
    nrows = psum.shape[1]
    pos_delta = jnp.concatenate(
        [psum[0:3, :], jnp.zeros((5, nrows), jnp.float32)], axis=0)
    posp_out_ref[...] = posp + pos_delta / jnp.maximum(cnt, 1.0)
    wn = wn1_ref[...]
    nh = x @ wn[:DN] + aggm @ wn[DN:] + bn1_ref[...]
    xo = _gelu(nh) @ wn2_ref[...] + bn2_ref[...]
    x_out_ref[...] = x + xo


def _node_mlp(x, posp, aggm, aggp, wn1, bn1, wn2, bn2):
    n = x.shape[0]
    return pl.pallas_call(
        _node_body,
        out_shape=[jax.ShapeDtypeStruct((n, DN), jnp.float32),
                   jax.ShapeDtypeStruct((8, n), jnp.float32)],
    )(x, posp, aggm, aggp, wn1, bn1, wn2, bn2)


def kernel(x, pos, edge_index, edge_attr,
           We1, be1, We2, be2, Wp1, bp1, Wp2, bp2, Wn1, bn1, Wn2, bn2):
    n = x.shape[0]
    src = edge_index[0].astype(jnp.int32)
    dst = edge_index[1].astype(jnp.int32)
    posp = jnp.concatenate(
        [jnp.transpose(pos), jnp.zeros((5, n), jnp.float32)], axis=0)
    zrow = jnp.zeros((16, DN), jnp.float32)
    zplane = jnp.zeros((4 * n,), jnp.float32)
    src2 = src.reshape(-1, GSUB, KCH)
    dst2 = dst.reshape(-1, GSUB, KCH)
    for l in range(NLAYERS):
        gxs, gxd, r = _sc_gather(x, posp, src2, dst2)
        m, p = _edge_mlp(gxs, gxd, r, edge_attr,
                         We1[l].astype(jnp.bfloat16), be1[l].reshape(1, -1),
                         We2[l].astype(jnp.bfloat16), be2[l].reshape(1, -1),
                         Wp1[l].astype(jnp.bfloat16), bp1[l].reshape(1, -1),
                         Wp2[l].reshape(1, -1), bp2[l].reshape(1, 1))
        aggm = _sc_scatter_m(m, dst, zrow, n)
        aggp = _sc_scatter_p(p, dst, zplane, n)
        x, posp = _node_mlp(x, posp, aggm, aggp.reshape(NW, 4, n),
                            Wn1[l], bn1[l].reshape(1, -1),
                            Wn2[l], bn2[l].reshape(1, -1))
    return x


# speedup vs baseline: 1.0416x; 1.0416x over previous
import functools

import jax
import jax.numpy as jnp
from jax import lax
from jax.experimental import pallas as pl
from jax.experimental.pallas import tpu as pltpu
from jax.experimental.pallas import tpu_sc as plsc

DN = 128
NLAYERS = 4

BE = 2560

NC = 2
NS = 16
NW = NC * NS
KCH = 128

GCH = 640
GSUB = GCH // KCH


def _gelu(v):
    return jax.nn.gelu(v, approximate=True)


def _sc_gather(xt, posp, src2, dst2):
    n = xt.shape[0]
    e = src2.shape[0] * src2.shape[1] * src2.shape[2]
    nch = e // GCH
    iters = (nch + NW - 1) // NW
    mesh = plsc.VectorSubcoreMesh(core_axis_name="c", subcore_axis_name="s")

    def body(xt_hbm, posp_hbm, src_hbm, dst_hbm,
             gxs_hbm, gxd_hbm, r_hbm,
             idx_s, idx_d, rows_s0, rows_s1, rows_d0, rows_d1,
             rbuf, px_v, py_v, pz_v,
             sem_s0, sem_s1, sem_d0, sem_d1):
        wid = lax.axis_index("s") * NC + lax.axis_index("c")
        rows_s = (rows_s0, rows_s1)
        rows_d = (rows_d0, rows_d1)
        sem_s = (sem_s0, sem_s1)
        sem_d = (sem_d0, sem_d1)
        pltpu.sync_copy(posp_hbm.at[0], px_v)
        pltpu.sync_copy(posp_hbm.at[1], py_v)
        pltpu.sync_copy(posp_hbm.at[2], pz_v)
        for r in range(4, 8):
            for j in range(GCH // 16):
                rbuf[r, pl.ds(j * 16, 16)] = jnp.zeros((16,), jnp.float32)

        def step(i, carry):
            c = wid + i * NW

            @pl.when(c < nch)
            def _():
                base = c * GCH
                pltpu.sync_copy(src_hbm.at[c], idx_s)
                pltpu.sync_copy(dst_hbm.at[c], idx_d)
                descs = [None, None]

                def _drain(sub):
                    b = sub % 2
                    descs[b][0].wait()
                    descs[b][1].wait()
                    pltpu.sync_copy(
                        rows_s[b], gxs_hbm.at[pl.ds(base + sub * KCH, KCH)])
                    pltpu.sync_copy(
                        rows_d[b], gxd_hbm.at[pl.ds(base + sub * KCH, KCH)])

                for sub in range(GSUB):
                    b = sub % 2
                    cs = pltpu.async_copy(xt_hbm.at[idx_s.at[sub]],
                                          rows_s[b], sem_s[b])
                    cd = pltpu.async_copy(xt_hbm.at[idx_d.at[sub]],
                                          rows_d[b], sem_d[b])
                    descs[b] = (cs, cd)
                    for j in range(KCH // 16):
                        col = sub * KCH + j * 16
                        iv_s = idx_s[sub, pl.ds(j * 16, 16)]
                        iv_d = idx_d[sub, pl.ds(j * 16, 16)]
                        relx = (plsc.load_gather(px_v, [iv_s])
                                - plsc.load_gather(px_v, [iv_d]))
                        rely = (plsc.load_gather(py_v, [iv_s])
                                - plsc.load_gather(py_v, [iv_d]))
                        relz = (plsc.load_gather(pz_v, [iv_s])
                                - plsc.load_gather(pz_v, [iv_d]))
                        d2 = relx * relx + rely * rely + relz * relz
                        rbuf[0, pl.ds(col, 16)] = relx
                        rbuf[1, pl.ds(col, 16)] = rely
                        rbuf[2, pl.ds(col, 16)] = relz
                        rbuf[3, pl.ds(col, 16)] = d2
                    if sub > 0:
                        _drain(sub - 1)
                _drain(GSUB - 1)
                pltpu.sync_copy(rbuf, r_hbm.at[:, pl.ds(base, GCH)])
            return carry

        lax.fori_loop(0, iters, step, 0)

    f = pl.kernel(
        body,
        out_type=[jax.ShapeDtypeStruct((e, DN), jnp.float32),
                  jax.ShapeDtypeStruct((e, DN), jnp.float32),
                  jax.ShapeDtypeStruct((8, e), jnp.float32)],
        mesh=mesh,
        compiler_params=pltpu.CompilerParams(needs_layout_passes=False),
        scratch_types=[pltpu.VMEM((GSUB, KCH), jnp.int32),
                       pltpu.VMEM((GSUB, KCH), jnp.int32),
                       pltpu.VMEM((KCH, DN), jnp.float32),
                       pltpu.VMEM((KCH, DN), jnp.float32),
                       pltpu.VMEM((KCH, DN), jnp.float32),
                       pltpu.VMEM((KCH, DN), jnp.float32),
                       pltpu.VMEM((8, GCH), jnp.float32),
                       pltpu.VMEM((n,), jnp.float32),
                       pltpu.VMEM((n,), jnp.float32),
                       pltpu.VMEM((n,), jnp.float32),
                       pltpu.SemaphoreType.DMA,
                       pltpu.SemaphoreType.DMA,
                       pltpu.SemaphoreType.DMA,
                       pltpu.SemaphoreType.DMA])
    return f(xt, posp, src2, dst2)


def _sc_scatter_m(m, dsti, zrow, n):
    e = m.shape[0]
    nch = e // KCH
    iters = (nch + NW - 1) // NW
    nun = n // 16
    zit = (nun + NS - 1) // NS
    mesh = plsc.VectorSubcoreMesh(core_axis_name="c", subcore_axis_name="s")

    def body(m_hbm, dst_hbm, zr_hbm, aggm_hbm,
             idx_v0, idx_v1, rows_v0, rows_v1, zb, accm,
             sem_i0, sem_i1, sem_r0, sem_r1):
        cid = lax.axis_index("c")
        sid = lax.axis_index("s")
        wid = sid * NC + cid
        idx_v = (idx_v0, idx_v1)
        rows_v = (rows_v0, rows_v1)
        sem_i = (sem_i0, sem_i1)
        sem_r = (sem_r0, sem_r1)

        pltpu.sync_copy(zr_hbm, zb)

        def zstep(k, carry):
            u = sid + k * NS

            @pl.when(u < nun)
            def _():
                pltpu.sync_copy(zb, accm.at[pl.ds(u * 16, 16)])
            return carry

        lax.fori_loop(0, zit, zstep, 0)
        plsc.subcore_barrier()

        def step(i, carry):
            descs = [None, None]
            cs = [None, None]
            for h in range(2):
                cs[h] = wid + (2 * i + h) * NW
                cc = jnp.minimum(cs[h], nch - 1)
                base = pl.multiple_of(cc * KCH, 8)
                ci = pltpu.async_copy(
                    dst_hbm.at[pl.ds(base, KCH)], idx_v[h], sem_i[h])
                cr = pltpu.async_copy(
                    m_hbm.at[pl.ds(base, KCH)], rows_v[h], sem_r[h])
                descs[h] = (ci, cr)
            for h in range(2):
                descs[h][0].wait()
                descs[h][1].wait()

                @pl.when(cs[h] < nch)
                def _(h=h):
                    pltpu.sync_copy(rows_v[h], accm.at[idx_v[h]], add=True)
            return carry

        lax.fori_loop(0, (iters + 1) // 2, step, 0)
        plsc.subcore_barrier()

        def ostep(k, carry):
            u = sid + k * NS

            @pl.when(u < nun)
            def _():
                pltpu.sync_copy(accm.at[pl.ds(u * 16, 16)],
                                aggm_hbm.at[cid, pl.ds(u * 16, 16)])
            return carry

        lax.fori_loop(0, zit, ostep, 0)

    f = pl.kernel(
        body,
        out_type=jax.ShapeDtypeStruct((NC, n, DN), jnp.float32),
        mesh=mesh,
        compiler_params=pltpu.CompilerParams(needs_layout_passes=False),
        scratch_types=[pltpu.VMEM((KCH,), jnp.int32),
                       pltpu.VMEM((KCH,), jnp.int32),
                       pltpu.VMEM((KCH, DN), jnp.float32),
                       pltpu.VMEM((KCH, DN), jnp.float32),
                       pltpu.VMEM((16, DN), jnp.float32),
                       pltpu.VMEM_SHARED((n, DN), jnp.float32),
                       pltpu.SemaphoreType.DMA,
                       pltpu.SemaphoreType.DMA,
                       pltpu.SemaphoreType.DMA,
                       pltpu.SemaphoreType.DMA])
    return f(m, dsti, zrow)


def _sc_scatter_p(p, dst2, zplane, n):
    e = p.shape[1]
    nch = e // GCH
    iters = (nch + NW - 1) // NW
    mesh = plsc.VectorSubcoreMesh(core_axis_name="c", subcore_axis_name="s")

    def body(p_hbm, dst_hbm, zp_hbm, aggp_hbm,
             idx_v0, idx_v1, pbuf0, pbuf1, accp,
             sem_i0, sem_i1, sem_p0, sem_p1):
        cid = lax.axis_index("c")
        sid = lax.axis_index("s")
        wid = sid * NC + cid
        idx_v = (idx_v0, idx_v1)
        pbuf = (pbuf0, pbuf1)
        sem_i = (sem_i0, sem_i1)
        sem_p = (sem_p0, sem_p1)

        pltpu.sync_copy(zp_hbm, accp)

        def step(i, carry):
            descs = [None, None]
            cs = [None, None]
            for h in range(2):
                cs[h] = wid + (2 * i + h) * NW
                cc = jnp.minimum(cs[h], nch - 1)
                base = pl.multiple_of(cc * GCH, 128)
                ci = pltpu.async_copy(
                    dst_hbm.at[cc], idx_v[h], sem_i[h])
                cp = pltpu.async_copy(
                    p_hbm.at[:, pl.ds(base, GCH)], pbuf[h], sem_p[h])
                descs[h] = (ci, cp)
            for h in range(2):
                descs[h][0].wait()
                descs[h][1].wait()

                @pl.when(cs[h] < nch)
                def _(h=h):
                    for j in range(GCH // 16):
                        iv = idx_v[h][j // 8, pl.ds((j % 8) * 16, 16)]
                        for k in range(4):
                            v = pbuf[h][k, pl.ds(j * 16, 16)]
                            plsc.addupdate_scatter(accp, [iv + k * n], v)
            return carry

        lax.fori_loop(0, (iters + 1) // 2, step, 0)
        pltpu.sync_copy(accp, aggp_hbm.at[wid])

    f = pl.kernel(
        body,
        out_type=jax.ShapeDtypeStruct((NW, 4 * n), jnp.float32),
        mesh=mesh,
        compiler_params=pltpu.CompilerParams(needs_layout_passes=False),
        scratch_types=[pltpu.VMEM((GSUB, KCH), jnp.int32),
                       pltpu.VMEM((GSUB, KCH), jnp.int32),
                       pltpu.VMEM((8, GCH), jnp.float32),
                       pltpu.VMEM((8, GCH), jnp.float32),
                       pltpu.VMEM((4 * n,), jnp.float32),
                       pltpu.SemaphoreType.DMA,
                       pltpu.SemaphoreType.DMA,
                       pltpu.SemaphoreType.DMA,
                       pltpu.SemaphoreType.DMA])
    return f(p, dst2, zplane)


def _edge_body(gxs_ref, gxd_ref, r_ref, ea_ref,
               we1_ref, be1_ref, we2_ref, be2_ref,
               wp1_ref, bp1_ref, wp2_ref, bp2_ref,
               m_out_ref, p_out_ref):
    xs = gxs_ref[...]
    xd = gxd_ref[...]
    relt = jnp.transpose(r_ref[...])
    d2 = relt[:, 3:4]
    s = jnp.concatenate([d2, d2 * 0.5, d2 * 0.25, d2 * 0.125], axis=1)
    fe = jnp.concatenate([jnp.sin(s), jnp.cos(s)], axis=1)
    w = we1_ref[...]
    bf = jnp.bfloat16
    f32 = jnp.float32
    dot = functools.partial(jnp.dot, preferred_element_type=f32)
    pre = (dot(xd.astype(bf), w[0:DN])
           + dot(xs.astype(bf), w[DN:2 * DN])
           + dot(fe.astype(bf), w[2 * DN:2 * DN + 8])
           + dot(ea_ref[...].astype(bf), w[2 * DN + 8:])
           + be1_ref[...])
    m1 = _gelu(pre)
    m = _gelu(dot(m1.astype(bf), we2_ref[...]) + be2_ref[...])
    p1 = _gelu(dot(m.astype(bf), wp1_ref[...]) + bp1_ref[...])
    p = jnp.sum(p1 * wp2_ref[...], axis=1, keepdims=True) + bp2_ref[...]
    nrows = xs.shape[0]
    pm = jnp.concatenate([relt[:, 0:3] * p,
                          jnp.ones((nrows, 1), jnp.float32),
                          jnp.zeros((nrows, 4), jnp.float32)], axis=1)
    m_out_ref[...] = m
    p_out_ref[...] = jnp.transpose(pm)


def _edge_mlp(gxs, gxd, r, ea, we1, be1, we2, be2, wp1, bp1, wp2r, bp2r):
    e = gxs.shape[0]
    full = lambda a: pl.BlockSpec(a.shape, lambda i: (0,) * a.ndim)
    blk = lambda w: pl.BlockSpec((BE, w), lambda i: (i, 0))
    pblk = pl.BlockSpec((8, BE), lambda i: (0, i))
    return pl.pallas_call(
        _edge_body,
        grid=(e // BE,),
        in_specs=[blk(DN), blk(DN), pblk, blk(16),
                  full(we1), full(be1), full(we2), full(be2),
                  full(wp1), full(bp1), full(wp2r), full(bp2r)],
        out_specs=[blk(DN), pblk],
        out_shape=[jax.ShapeDtypeStruct((e, DN), jnp.float32),
                   jax.ShapeDtypeStruct((8, e), jnp.float32)],
    )(gxs, gxd, r, ea, we1, be1, we2, be2, wp1, bp1, wp2r, bp2r)


def _node_body(x_ref, posp_ref, aggm_ref, aggp_ref,
               wn1_ref, bn1_ref, wn2_ref, bn2_ref,
               x_out_ref, posp_out_ref):
    x = x_ref[...]
    posp = posp_ref[...]
    aggm = jnp.sum(aggm_ref[...], axis=0)
    psum = jnp.sum(aggp_ref[...], axis=0)
    cnt = psum[3:4, :]
    nrows = psum.shape[1]
    pos_delta = jnp.concatenate(
        [psum[0:3, :], jnp.zeros((5, nrows), jnp.float32)], axis=0)
    posp_out_ref[...] = posp + pos_delta / jnp.maximum(cnt, 1.0)
    wn = wn1_ref[...]
    nh = x @ wn[:DN] + aggm @ wn[DN:] + bn1_ref[...]
    xo = _gelu(nh) @ wn2_ref[...] + bn2_ref[...]
    x_out_ref[...] = x + xo


def _node_mlp(x, posp, aggm, aggp, wn1, bn1, wn2, bn2):
    n = x.shape[0]
    return pl.pallas_call(
        _node_body,
        out_shape=[jax.ShapeDtypeStruct((n, DN), jnp.float32),
                   jax.ShapeDtypeStruct((8, n), jnp.float32)],
    )(x, posp, aggm, aggp, wn1, bn1, wn2, bn2)


def kernel(x, pos, edge_index, edge_attr,
           We1, be1, We2, be2, Wp1, bp1, Wp2, bp2, Wn1, bn1, Wn2, bn2):
    n = x.shape[0]
    src = edge_index[0].astype(jnp.int32)
    dst = edge_index[1].astype(jnp.int32)
    posp = jnp.concatenate(
        [jnp.transpose(pos), jnp.zeros((5, n), jnp.float32)], axis=0)
    zrow = jnp.zeros((16, DN), jnp.float32)
    zplane = jnp.zeros((4 * n,), jnp.float32)
    src2 = src.reshape(-1, GSUB, KCH)
    dst2 = dst.reshape(-1, GSUB, KCH)
    for l in range(NLAYERS):
        gxs, gxd, r = _sc_gather(x, posp, src2, dst2)
        m, p = _edge_mlp(gxs, gxd, r, edge_attr,
                         We1[l].astype(jnp.bfloat16), be1[l].reshape(1, -1),
                         We2[l].astype(jnp.bfloat16), be2[l].reshape(1, -1),
                         Wp1[l].astype(jnp.bfloat16), bp1[l].reshape(1, -1),
                         Wp2[l].reshape(1, -1), bp2[l].reshape(1, 1))
        aggm = _sc_scatter_m(m, dst, zrow, n)
        aggp = _sc_scatter_p(p, dst2, zplane, n)
        x, posp = _node_mlp(x, posp, aggm, aggp.reshape(NW, 4, n),
                            Wn1[l], bn1[l].reshape(1, -1),
                            Wn2[l], bn2[l].reshape(1, -1))
    return x


# speedup vs baseline: 1.0809x; 1.0377x over previous
import functools

import jax
import jax.numpy as jnp
from jax import lax
from jax.experimental import pallas as pl
from jax.experimental.pallas import tpu as pltpu
from jax.experimental.pallas import tpu_sc as plsc

DN = 128
NLAYERS = 4

BE = 1280

NC = 2
NS = 16
NW = NC * NS
KCH = 128

GCH = 640
GSUB = GCH // KCH


def _gelu(v):
    return jax.nn.gelu(v, approximate=True)


def _sc_gather(xt, posp, src2, dst2):
    n = xt.shape[0]
    e = src2.shape[0] * src2.shape[1] * src2.shape[2]
    nch = e // GCH
    iters = (nch + NW - 1) // NW
    mesh = plsc.VectorSubcoreMesh(core_axis_name="c", subcore_axis_name="s")

    def body(xt_hbm, posp_hbm, src_hbm, dst_hbm,
             gxs_hbm, gxd_hbm, r_hbm,
             idx_s, idx_d, rows_s0, rows_s1, rows_d0, rows_d1,
             rbuf, px_v, py_v, pz_v,
             sem_s0, sem_s1, sem_d0, sem_d1):
        wid = lax.axis_index("s") * NC + lax.axis_index("c")
        rows_s = (rows_s0, rows_s1)
        rows_d = (rows_d0, rows_d1)
        sem_s = (sem_s0, sem_s1)
        sem_d = (sem_d0, sem_d1)
        pltpu.sync_copy(posp_hbm.at[0], px_v)
        pltpu.sync_copy(posp_hbm.at[1], py_v)
        pltpu.sync_copy(posp_hbm.at[2], pz_v)
        for r in range(4, 8):
            for j in range(GCH // 16):
                rbuf[r, pl.ds(j * 16, 16)] = jnp.zeros((16,), jnp.float32)

        def step(i, carry):
            c = wid + i * NW

            @pl.when(c < nch)
            def _():
                base = c * GCH
                pltpu.sync_copy(src_hbm.at[c], idx_s)
                pltpu.sync_copy(dst_hbm.at[c], idx_d)
                descs = [None, None]

                def _drain(sub):
                    b = sub % 2
                    descs[b][0].wait()
                    descs[b][1].wait()
                    pltpu.sync_copy(
                        rows_s[b], gxs_hbm.at[pl.ds(base + sub * KCH, KCH)])
                    pltpu.sync_copy(
                        rows_d[b], gxd_hbm.at[pl.ds(base + sub * KCH, KCH)])

                for sub in range(GSUB):
                    b = sub % 2
                    cs = pltpu.async_copy(xt_hbm.at[idx_s.at[sub]],
                                          rows_s[b], sem_s[b])
                    cd = pltpu.async_copy(xt_hbm.at[idx_d.at[sub]],
                                          rows_d[b], sem_d[b])
                    descs[b] = (cs, cd)
                    for j in range(KCH // 16):
                        col = sub * KCH + j * 16
                        iv_s = idx_s[sub, pl.ds(j * 16, 16)]
                        iv_d = idx_d[sub, pl.ds(j * 16, 16)]
                        relx = (plsc.load_gather(px_v, [iv_s])
                                - plsc.load_gather(px_v, [iv_d]))
                        rely = (plsc.load_gather(py_v, [iv_s])
                                - plsc.load_gather(py_v, [iv_d]))
                        relz = (plsc.load_gather(pz_v, [iv_s])
                                - plsc.load_gather(pz_v, [iv_d]))
                        d2 = relx * relx + rely * rely + relz * relz
                        rbuf[0, pl.ds(col, 16)] = relx
                        rbuf[1, pl.ds(col, 16)] = rely
                        rbuf[2, pl.ds(col, 16)] = relz
                        rbuf[3, pl.ds(col, 16)] = d2
                    if sub > 0:
                        _drain(sub - 1)
                _drain(GSUB - 1)
                pltpu.sync_copy(rbuf, r_hbm.at[:, pl.ds(base, GCH)])
            return carry

        lax.fori_loop(0, iters, step, 0)

    f = pl.kernel(
        body,
        out_type=[jax.ShapeDtypeStruct((e, DN), jnp.float32),
                  jax.ShapeDtypeStruct((e, DN), jnp.float32),
                  jax.ShapeDtypeStruct((8, e), jnp.float32)],
        mesh=mesh,
        compiler_params=pltpu.CompilerParams(needs_layout_passes=False),
        scratch_types=[pltpu.VMEM((GSUB, KCH), jnp.int32),
                       pltpu.VMEM((GSUB, KCH), jnp.int32),
                       pltpu.VMEM((KCH, DN), jnp.float32),
                       pltpu.VMEM((KCH, DN), jnp.float32),
                       pltpu.VMEM((KCH, DN), jnp.float32),
                       pltpu.VMEM((KCH, DN), jnp.float32),
                       pltpu.VMEM((8, GCH), jnp.float32),
                       pltpu.VMEM((n,), jnp.float32),
                       pltpu.VMEM((n,), jnp.float32),
                       pltpu.VMEM((n,), jnp.float32),
                       pltpu.SemaphoreType.DMA,
                       pltpu.SemaphoreType.DMA,
                       pltpu.SemaphoreType.DMA,
                       pltpu.SemaphoreType.DMA])
    return f(xt, posp, src2, dst2)


def _sc_scatter_m(m, dsti, zrow, n):
    e = m.shape[0]
    nch = e // KCH
    iters = (nch + NW - 1) // NW
    nun = n // 16
    zit = (nun + NS - 1) // NS
    mesh = plsc.VectorSubcoreMesh(core_axis_name="c", subcore_axis_name="s")

    def body(m_hbm, dst_hbm, zr_hbm, aggm_hbm,
             idx_v0, idx_v1, rows_v0, rows_v1, zb, accm,
             sem_i0, sem_i1, sem_r0, sem_r1):
        cid = lax.axis_index("c")
        sid = lax.axis_index("s")
        wid = sid * NC + cid
        idx_v = (idx_v0, idx_v1)
        rows_v = (rows_v0, rows_v1)
        sem_i = (sem_i0, sem_i1)
        sem_r = (sem_r0, sem_r1)

        pltpu.sync_copy(zr_hbm, zb)

        def zstep(k, carry):
            u = sid + k * NS

            @pl.when(u < nun)
            def _():
                pltpu.sync_copy(zb, accm.at[pl.ds(u * 16, 16)])
            return carry

        lax.fori_loop(0, zit, zstep, 0)
        plsc.subcore_barrier()

        def step(i, carry):
            descs = [None, None]
            cs = [None, None]
            for h in range(2):
                cs[h] = wid + (2 * i + h) * NW
                cc = jnp.minimum(cs[h], nch - 1)
                base = pl.multiple_of(cc * KCH, 8)
                ci = pltpu.async_copy(
                    dst_hbm.at[pl.ds(base, KCH)], idx_v[h], sem_i[h])
                cr = pltpu.async_copy(
                    m_hbm.at[pl.ds(base, KCH)], rows_v[h], sem_r[h])
                descs[h] = (ci, cr)
            for h in range(2):
                descs[h][0].wait()
                descs[h][1].wait()

                @pl.when(cs[h] < nch)
                def _(h=h):
                    pltpu.sync_copy(rows_v[h], accm.at[idx_v[h]], add=True)
            return carry

        lax.fori_loop(0, (iters + 1) // 2, step, 0)
        plsc.subcore_barrier()

        def ostep(k, carry):
            u = sid + k * NS

            @pl.when(u < nun)
            def _():
                pltpu.sync_copy(accm.at[pl.ds(u * 16, 16)],
                                aggm_hbm.at[cid, pl.ds(u * 16, 16)])
            return carry

        lax.fori_loop(0, zit, ostep, 0)

    f = pl.kernel(
        body,
        out_type=jax.ShapeDtypeStruct((NC, n, DN), jnp.float32),
        mesh=mesh,
        compiler_params=pltpu.CompilerParams(needs_layout_passes=False),
        scratch_types=[pltpu.VMEM((KCH,), jnp.int32),
                       pltpu.VMEM((KCH,), jnp.int32),
                       pltpu.VMEM((KCH, DN), jnp.float32),
                       pltpu.VMEM((KCH, DN), jnp.float32),
                       pltpu.VMEM((16, DN), jnp.float32),
                       pltpu.VMEM_SHARED((n, DN), jnp.float32),
                       pltpu.SemaphoreType.DMA,
                       pltpu.SemaphoreType.DMA,
                       pltpu.SemaphoreType.DMA,
                       pltpu.SemaphoreType.DMA])
    return f(m, dsti, zrow)


def _sc_scatter_p(p, dst2, zplane, n):
    e = p.shape[1]
    nch = e // GCH
    iters = (nch + NW - 1) // NW
    mesh = plsc.VectorSubcoreMesh(core_axis_name="c", subcore_axis_name="s")

    def body(p_hbm, dst_hbm, zp_hbm, aggp_hbm,
             idx_v0, idx_v1, pbuf0, pbuf1, accp,
             sem_i0, sem_i1, sem_p0, sem_p1):
        cid = lax.axis_index("c")
        sid = lax.axis_index("s")
        wid = sid * NC + cid
        idx_v = (idx_v0, idx_v1)
        pbuf = (pbuf0, pbuf1)
        sem_i = (sem_i0, sem_i1)
        sem_p = (sem_p0, sem_p1)

        pltpu.sync_copy(zp_hbm, accp)

        def step(i, carry):
            descs = [None, None]
            cs = [None, None]
            for h in range(2):
                cs[h] = wid + (2 * i + h) * NW
                cc = jnp.minimum(cs[h], nch - 1)
                base = pl.multiple_of(cc * GCH, 128)
                ci = pltpu.async_copy(
                    dst_hbm.at[cc], idx_v[h], sem_i[h])
                cp = pltpu.async_copy(
                    p_hbm.at[:, pl.ds(base, GCH)], pbuf[h], sem_p[h])
                descs[h] = (ci, cp)
            for h in range(2):
                descs[h][0].wait()
                descs[h][1].wait()

                @pl.when(cs[h] < nch)
                def _(h=h):
                    for j in range(GCH // 16):
                        iv = idx_v[h][j // 8, pl.ds((j % 8) * 16, 16)]
                        for k in range(4):
                            v = pbuf[h][k, pl.ds(j * 16, 16)]
                            plsc.addupdate_scatter(accp, [iv + k * n], v)
            return carry

        lax.fori_loop(0, (iters + 1) // 2, step, 0)
        pltpu.sync_copy(accp, aggp_hbm.at[wid])

    f = pl.kernel(
        body,
        out_type=jax.ShapeDtypeStruct((NW, 4 * n), jnp.float32),
        mesh=mesh,
        compiler_params=pltpu.CompilerParams(needs_layout_passes=False),
        scratch_types=[pltpu.VMEM((GSUB, KCH), jnp.int32),
                       pltpu.VMEM((GSUB, KCH), jnp.int32),
                       pltpu.VMEM((8, GCH), jnp.float32),
                       pltpu.VMEM((8, GCH), jnp.float32),
                       pltpu.VMEM((4 * n,), jnp.float32),
                       pltpu.SemaphoreType.DMA,
                       pltpu.SemaphoreType.DMA,
                       pltpu.SemaphoreType.DMA,
                       pltpu.SemaphoreType.DMA])
    return f(p, dst2, zplane)


def _edge_body(gxs_ref, gxd_ref, r_ref, ea_ref,
               we1_ref, be1_ref, we2_ref, be2_ref,
               wp1_ref, bp1_ref, wp2_ref, bp2_ref,
               m_out_ref, p_out_ref):
    xs = gxs_ref[...]
    xd = gxd_ref[...]
    relt = jnp.transpose(r_ref[...])
    d2 = relt[:, 3:4]
    s = jnp.concatenate([d2, d2 * 0.5, d2 * 0.25, d2 * 0.125], axis=1)
    fe = jnp.concatenate([jnp.sin(s), jnp.cos(s)], axis=1)
    w = we1_ref[...]
    bf = jnp.bfloat16
    f32 = jnp.float32
    dot = functools.partial(jnp.dot, preferred_element_type=f32)
    pre = (dot(xd.astype(bf), w[0:DN])
           + dot(xs.astype(bf), w[DN:2 * DN])
           + dot(fe.astype(bf), w[2 * DN:2 * DN + 8])
           + dot(ea_ref[...].astype(bf), w[2 * DN + 8:])
           + be1_ref[...])
    m1 = _gelu(pre)
    m = _gelu(dot(m1.astype(bf), we2_ref[...]) + be2_ref[...])
    p1 = _gelu(dot(m.astype(bf), wp1_ref[...]) + bp1_ref[...])
    p = jnp.sum(p1 * wp2_ref[...], axis=1, keepdims=True) + bp2_ref[...]
    nrows = xs.shape[0]
    pm = jnp.concatenate([relt[:, 0:3] * p,
                          jnp.ones((nrows, 1), jnp.float32),
                          jnp.zeros((nrows, 4), jnp.float32)], axis=1)
    m_out_ref[...] = m
    p_out_ref[...] = jnp.transpose(pm)


def _edge_mlp(gxs, gxd, r, ea, we1, be1, we2, be2, wp1, bp1, wp2r, bp2r):
    e = gxs.shape[0]
    full = lambda a: pl.BlockSpec(a.shape, lambda i: (0,) * a.ndim)
    blk = lambda w: pl.BlockSpec((BE, w), lambda i: (i, 0))
    pblk = pl.BlockSpec((8, BE), lambda i: (0, i))
    return pl.pallas_call(
        _edge_body,
        grid=(e // BE,),
        in_specs=[blk(DN), blk(DN), pblk, blk(16),
                  full(we1), full(be1), full(we2), full(be2),
                  full(wp1), full(bp1), full(wp2r), full(bp2r)],
        out_specs=[blk(DN), pblk],
        out_shape=[jax.ShapeDtypeStruct((e, DN), jnp.float32),
                   jax.ShapeDtypeStruct((8, e), jnp.float32)],
    )(gxs, gxd, r, ea, we1, be1, we2, be2, wp1, bp1, wp2r, bp2r)


def _node_body(x_ref, posp_ref, aggm_ref, aggp_ref,
               wn1_ref, bn1_ref, wn2_ref, bn2_ref,
               x_out_ref, posp_out_ref):
    x = x_ref[...]
    posp = posp_ref[...]
    aggm = jnp.sum(aggm_ref[...], axis=0)
    psum = jnp.sum(aggp_ref[...], axis=0)
    cnt = psum[3:4, :]
    nrows = psum.shape[1]
    pos_delta = jnp.concatenate(
        [psum[0:3, :], jnp.zeros((5, nrows), jnp.float32)], axis=0)
    posp_out_ref[...] = posp + pos_delta / jnp.maximum(cnt, 1.0)
    wn = wn1_ref[...]
    nh = x @ wn[:DN] + aggm @ wn[DN:] + bn1_ref[...]
    xo = _gelu(nh) @ wn2_ref[...] + bn2_ref[...]
    x_out_ref[...] = x + xo


def _node_mlp(x, posp, aggm, aggp, wn1, bn1, wn2, bn2):
    n = x.shape[0]
    return pl.pallas_call(
        _node_body,
        out_shape=[jax.ShapeDtypeStruct((n, DN), jnp.float32),
                   jax.ShapeDtypeStruct((8, n), jnp.float32)],
    )(x, posp, aggm, aggp, wn1, bn1, wn2, bn2)


def kernel(x, pos, edge_index, edge_attr,
           We1, be1, We2, be2, Wp1, bp1, Wp2, bp2, Wn1, bn1, Wn2, bn2):
    n = x.shape[0]
    src = edge_index[0].astype(jnp.int32)
    dst = edge_index[1].astype(jnp.int32)
    posp = jnp.concatenate(
        [jnp.transpose(pos), jnp.zeros((5, n), jnp.float32)], axis=0)
    zrow = jnp.zeros((16, DN), jnp.float32)
    zplane = jnp.zeros((4 * n,), jnp.float32)
    src2 = src.reshape(-1, GSUB, KCH)
    dst2 = dst.reshape(-1, GSUB, KCH)
    for l in range(NLAYERS):
        gxs, gxd, r = _sc_gather(x, posp, src2, dst2)
        m, p = _edge_mlp(gxs, gxd, r, edge_attr,
                         We1[l].astype(jnp.bfloat16), be1[l].reshape(1, -1),
                         We2[l].astype(jnp.bfloat16), be2[l].reshape(1, -1),
                         Wp1[l].astype(jnp.bfloat16), bp1[l].reshape(1, -1),
                         Wp2[l].reshape(1, -1), bp2[l].reshape(1, 1))
        aggm = _sc_scatter_m(m, dst, zrow, n)
        aggp = _sc_scatter_p(p, dst2, zplane, n)
        x, posp = _node_mlp(x, posp, aggm, aggp.reshape(NW, 4, n),
                            Wn1[l], bn1[l].reshape(1, -1),
                            Wn2[l], bn2[l].reshape(1, -1))
    return x
